# baseline (speedup 1.0000x reference)
import functools

import jax
import jax.numpy as jnp
from jax import lax
from jax.experimental import pallas as pl
from jax.experimental.pallas import tpu as pltpu
from jax.experimental.pallas import tpu_sc as plsc

N_NODES = 10000
N_EDGES = 320000
DIM = 128

NC = 2
NS = 16
NW = NC * NS
CHUNK = 128
NCHUNK = 80
EPT = CHUNK * NCHUNK
E_PAD = EPT * NW
PAD = E_PAD - N_EDGES
NCHUNK0 = NCHUNK
NCHUNK1 = 2 * NCHUNK - NCHUNK0
N_PAD = 10016
STRIPE = 632
ZLAST = N_PAD - 15 * STRIPE
OLAST = N_NODES - 15 * STRIPE


def _build_chunk_table(src1, dst2):
    def _row(j, _):
        for k in range(CHUNK // 16):
            dst2[j, pl.ds(16 * k, 16)] = src1[pl.ds(j * CHUNK + 16 * k, 16)]
        return 0

    lax.fori_loop(0, NCHUNK, _row, 0)


def _zero_stripe(sid, zbuf, dst_s):
    def _run(r0, nrows):
        for t in range(nrows // CHUNK):
            pltpu.sync_copy(zbuf, dst_s.at[pl.ds(r0 + t * CHUNK, CHUNK)])
        rem = nrows % CHUNK
        if rem:
            pltpu.sync_copy(
                zbuf.at[pl.ds(0, rem)],
                dst_s.at[pl.ds(r0 + (nrows // CHUNK) * CHUNK, rem)],
            )

    @pl.when(sid < 15)
    def _():
        _run(pl.multiple_of(sid * STRIPE, 8), STRIPE)

    @pl.when(sid == 15)
    def _():
        _run(15 * STRIPE, ZLAST)


def _copy_out_stripe(sid, cid, src_s, out_hbm):
    @pl.when(sid < 15)
    def _():
        o0 = pl.multiple_of(sid * STRIPE, 8)
        pltpu.sync_copy(src_s.at[pl.ds(o0, STRIPE)], out_hbm.at[cid, pl.ds(o0, STRIPE)])

    @pl.when(sid == 15)
    def _():
        pltpu.sync_copy(
            src_s.at[pl.ds(15 * STRIPE, OLAST)],
            out_hbm.at[cid, pl.ds(15 * STRIPE, OLAST)],
        )

_mesh = plsc.VectorSubcoreMesh(core_axis_name="c", subcore_axis_name="s")


@functools.partial(
    pl.kernel,
    mesh=_mesh,
    out_type=jax.ShapeDtypeStruct((NC, N_NODES, DIM), jnp.float32),
    scratch_types=[
        pltpu.VMEM((EPT,), jnp.int32),
        pltpu.VMEM((NCHUNK, CHUNK), jnp.int32),
        pltpu.VMEM((CHUNK, DIM), jnp.float32),
        pltpu.VMEM_SHARED((N_PAD, DIM), jnp.float32),
    ],
)
def _deg_kernel(col_hbm, out_hbm, stage_v, col_v, ones_v, deg_s):
    cid = lax.axis_index("c")
    sid = lax.axis_index("s")
    wid = sid * NC + cid

    pltpu.sync_copy(col_hbm.at[pl.ds(pl.multiple_of(wid * EPT, 8), EPT)], stage_v)
    _build_chunk_table(stage_v, col_v)

    def _fill(i, _):
        for k in range(DIM // 16):
            ones_v[i, pl.ds(16 * k, 16)] = jnp.zeros((16,), jnp.float32)
        return 0

    lax.fori_loop(0, CHUNK, _fill, 0)
    _zero_stripe(sid, ones_v, deg_s)

    def _fill1(i, _):
        for k in range(DIM // 16):
            ones_v[i, pl.ds(16 * k, 16)] = jnp.ones((16,), jnp.float32)
        return 0

    lax.fori_loop(0, CHUNK, _fill1, 0)
    plsc.subcore_barrier()

    def _body(j, _):
        pltpu.sync_copy(ones_v, deg_s.at[col_v.at[j]], add=True)
        return 0

    lax.fori_loop(0, NCHUNK, _body, 0)
    plsc.subcore_barrier()
    _copy_out_stripe(sid, cid, deg_s, out_hbm)


@functools.partial(
    pl.kernel,
    mesh=_mesh,
    out_type=jax.ShapeDtypeStruct((NC, N_NODES, DIM), jnp.float32),
    scratch_types=[
        pltpu.VMEM((CHUNK,), jnp.int32),
        pltpu.VMEM((CHUNK,), jnp.int32),
        pltpu.VMEM((CHUNK,), jnp.int32),
        pltpu.VMEM((CHUNK,), jnp.int32),
        pltpu.VMEM((CHUNK, DIM), jnp.float32),
        pltpu.VMEM((CHUNK, DIM), jnp.float32),
        pltpu.VMEM_SHARED((N_PAD, DIM), jnp.float32),
        pltpu.SemaphoreType.DMA,
        pltpu.SemaphoreType.DMA,
        pltpu.SemaphoreType.DMA,
        pltpu.SemaphoreType.DMA,
        pltpu.SemaphoreType.DMA,
    ],
)
def _scatter_kernel(
    row_hbm, col_hbm, y_hbm, out_hbm,
    row_a, col_a, row_b, col_b, buf_a, buf_b, acc_s,
    sem_ia, sem_ib, sem_ga, sem_gb, sem_sa,
):
    cid = lax.axis_index("c")
    sid = lax.axis_index("s")
    base = (sid * 2 * NCHUNK + cid * NCHUNK0) * CHUNK
    nch = jnp.where(cid == 0, NCHUNK0, NCHUNK1)

    def _zero(i, _):
        for k in range(DIM // 16):
            buf_a[i, pl.ds(16 * k, 16)] = jnp.zeros((16,), jnp.float32)
        return 0

    lax.fori_loop(0, CHUNK, _zero, 0)
    _zero_stripe(sid, buf_a, acc_s)
    plsc.subcore_barrier()

    def _start_idx(j, row_v, col_v, sem):
        off = base + j * CHUNK
        pltpu.async_copy(row_hbm.at[pl.ds(off, CHUNK)], row_v, sem)
        pltpu.async_copy(col_hbm.at[pl.ds(off, CHUNK)], col_v, sem)

    def _wait_idx(row_v, col_v, sem):
        pltpu.make_async_copy(row_hbm.at[pl.ds(0, CHUNK)], row_v, sem).wait()
        pltpu.make_async_copy(col_hbm.at[pl.ds(0, CHUNK)], col_v, sem).wait()

    def _wait_gather(buf, sem):
        pltpu.make_async_copy(y_hbm.at[pl.ds(0, CHUNK)], buf, sem).wait()

    _start_idx(0, row_a, col_a, sem_ia)
    _wait_idx(row_a, col_a, sem_ia)
    pltpu.async_copy(y_hbm.at[row_a], buf_a, sem_ga)
    _start_idx(1, row_b, col_b, sem_ib)

    def _pair(i, _):
        j0 = 2 * i
        j1 = j0 + 1
        _wait_idx(row_b, col_b, sem_ib)
        pltpu.async_copy(y_hbm.at[row_b], buf_b, sem_gb)
        _wait_gather(buf_a, sem_ga)
        pltpu.async_copy(buf_a, acc_s.at[col_a], sem_sa, add=True)

        _wait_gather(buf_b, sem_gb)
        pltpu.sync_copy(buf_b, acc_s.at[col_b], add=True)

        @pl.when(j0 + 2 < nch)
        def _():
            pltpu.make_async_copy(buf_a, acc_s.at[pl.ds(0, CHUNK)], sem_sa).wait()
            _start_idx(j0 + 2, row_a, col_a, sem_ia)
            _wait_idx(row_a, col_a, sem_ia)
            pltpu.async_copy(y_hbm.at[row_a], buf_a, sem_ga)

        @pl.when(j1 + 2 < nch)
        def _():
            _start_idx(j1 + 2, row_b, col_b, sem_ib)

        return 0

    lax.fori_loop(0, nch // 2, _pair, 0)
    pltpu.make_async_copy(buf_a, acc_s.at[pl.ds(0, CHUNK)], sem_sa).wait()
    plsc.subcore_barrier()
    _copy_out_stripe(sid, cid, acc_s, out_hbm)


def _mm_body(x_ref, w_ref, xw_ref):
    xw_ref[...] = jnp.dot(x_ref[...], w_ref[...], preferred_element_type=jnp.float32)


def _scale_body(xw_ref, degp_ref, y_ref):
    dinv = lax.rsqrt(degp_ref[0] + degp_ref[1] + 1.0)
    y_ref[...] = dinv * xw_ref[...]


def _combine_body(p_ref, y_ref, degp_ref, out_ref):
    dinv = lax.rsqrt(degp_ref[0] + degp_ref[1] + 1.0)
    out_ref[...] = dinv * (p_ref[0] + p_ref[1] + y_ref[...])


_MM_ROWS = 1000


def kernel(node_feature, edge_index, W):
    row = edge_index[0].astype(jnp.int32)
    col = edge_index[1].astype(jnp.int32)
    pad_idx = jnp.arange(PAD, dtype=jnp.int32)
    row_p = jnp.concatenate([row, pad_idx % N_NODES])
    col_p = jnp.concatenate([col, N_NODES + (pad_idx % (N_PAD - N_NODES))])

    deg_p = _deg_kernel(col_p)

    grid = N_NODES // _MM_ROWS
    xw = pl.pallas_call(
        _mm_body,
        grid=(grid,),
        in_specs=[
            pl.BlockSpec((_MM_ROWS, DIM), lambda i: (i, 0)),
            pl.BlockSpec((DIM, DIM), lambda i: (0, 0)),
        ],
        out_specs=pl.BlockSpec((_MM_ROWS, DIM), lambda i: (i, 0)),
        out_shape=jax.ShapeDtypeStruct((N_NODES, DIM), jnp.float32),
    )(node_feature, W)

    y = pl.pallas_call(
        _scale_body,
        grid=(grid,),
        in_specs=[
            pl.BlockSpec((_MM_ROWS, DIM), lambda i: (i, 0)),
            pl.BlockSpec((NC, _MM_ROWS, DIM), lambda i: (0, i, 0)),
        ],
        out_specs=pl.BlockSpec((_MM_ROWS, DIM), lambda i: (i, 0)),
        out_shape=jax.ShapeDtypeStruct((N_NODES, DIM), jnp.float32),
    )(xw, deg_p)

    parts = _scatter_kernel(row_p, col_p, y)

    out = pl.pallas_call(
        _combine_body,
        grid=(grid,),
        in_specs=[
            pl.BlockSpec((NC, _MM_ROWS, DIM), lambda i: (0, i, 0)),
            pl.BlockSpec((_MM_ROWS, DIM), lambda i: (i, 0)),
            pl.BlockSpec((NC, _MM_ROWS, DIM), lambda i: (0, i, 0)),
        ],
        out_specs=pl.BlockSpec((_MM_ROWS, DIM), lambda i: (i, 0)),
        out_shape=jax.ShapeDtypeStruct((N_NODES, DIM), jnp.float32),
    )(parts, y, deg_p)
    return out


# speedup vs baseline: 1.0667x; 1.0667x over previous
import functools

import jax
import jax.numpy as jnp
from jax import lax
from jax.experimental import pallas as pl
from jax.experimental.pallas import tpu as pltpu
from jax.experimental.pallas import tpu_sc as plsc

N_NODES = 10000
N_EDGES = 320000
DIM = 128

NC = 2
NS = 16
NW = NC * NS
CHUNK = 128
NCHUNK = 80
EPT = CHUNK * NCHUNK
E_PAD = EPT * NW
PAD = E_PAD - N_EDGES
NCHUNK0 = NCHUNK
NCHUNK1 = 2 * NCHUNK - NCHUNK0
N_PAD = 10016
STRIPE = 632
ZLAST = N_PAD - 15 * STRIPE
OLAST = N_NODES - 15 * STRIPE


def _build_chunk_table(src1, dst2):
    def _row(j, _):
        for k in range(CHUNK // 16):
            dst2[j, pl.ds(16 * k, 16)] = src1[pl.ds(j * CHUNK + 16 * k, 16)]
        return 0

    lax.fori_loop(0, NCHUNK, _row, 0)


def _zero_stripe(sid, zbuf, dst_s):
    def _run(r0, nrows):
        for t in range(nrows // CHUNK):
            pltpu.sync_copy(zbuf, dst_s.at[pl.ds(r0 + t * CHUNK, CHUNK)])
        rem = nrows % CHUNK
        if rem:
            pltpu.sync_copy(
                zbuf.at[pl.ds(0, rem)],
                dst_s.at[pl.ds(r0 + (nrows // CHUNK) * CHUNK, rem)],
            )

    @pl.when(sid < 15)
    def _():
        _run(pl.multiple_of(sid * STRIPE, 8), STRIPE)

    @pl.when(sid == 15)
    def _():
        _run(15 * STRIPE, ZLAST)


def _copy_out_stripe(sid, cid, src_s, out_hbm):
    @pl.when(sid < 15)
    def _():
        o0 = pl.multiple_of(sid * STRIPE, 8)
        pltpu.sync_copy(src_s.at[pl.ds(o0, STRIPE)], out_hbm.at[cid, pl.ds(o0, STRIPE)])

    @pl.when(sid == 15)
    def _():
        pltpu.sync_copy(
            src_s.at[pl.ds(15 * STRIPE, OLAST)],
            out_hbm.at[cid, pl.ds(15 * STRIPE, OLAST)],
        )

_mesh = plsc.VectorSubcoreMesh(core_axis_name="c", subcore_axis_name="s")


@functools.partial(
    pl.kernel,
    mesh=_mesh,
    out_type=jax.ShapeDtypeStruct((NC, N_NODES, DIM), jnp.float32),
    scratch_types=[
        pltpu.VMEM((EPT,), jnp.int32),
        pltpu.VMEM((NCHUNK, CHUNK), jnp.int32),
        pltpu.VMEM((CHUNK, DIM), jnp.float32),
        pltpu.VMEM_SHARED((N_PAD, DIM), jnp.float32),
    ],
)
def _deg_kernel(col_hbm, out_hbm, stage_v, col_v, ones_v, deg_s):
    cid = lax.axis_index("c")
    sid = lax.axis_index("s")
    wid = sid * NC + cid

    pltpu.sync_copy(col_hbm.at[pl.ds(pl.multiple_of(wid * EPT, 8), EPT)], stage_v)
    _build_chunk_table(stage_v, col_v)

    def _fill(i, _):
        for k in range(DIM // 16):
            ones_v[i, pl.ds(16 * k, 16)] = jnp.zeros((16,), jnp.float32)
        return 0

    lax.fori_loop(0, CHUNK, _fill, 0)
    _zero_stripe(sid, ones_v, deg_s)

    def _fill1(i, _):
        for k in range(DIM // 16):
            ones_v[i, pl.ds(16 * k, 16)] = jnp.ones((16,), jnp.float32)
        return 0

    lax.fori_loop(0, CHUNK, _fill1, 0)
    plsc.subcore_barrier()

    def _body(j, _):
        pltpu.sync_copy(ones_v, deg_s.at[col_v.at[j]], add=True)
        return 0

    lax.fori_loop(0, NCHUNK, _body, 0)
    plsc.subcore_barrier()
    _copy_out_stripe(sid, cid, deg_s, out_hbm)


@functools.partial(
    pl.kernel,
    mesh=_mesh,
    out_type=jax.ShapeDtypeStruct((NC, N_NODES, DIM), jnp.float32),
    scratch_types=[
        pltpu.VMEM((CHUNK,), jnp.int32),
        pltpu.VMEM((CHUNK,), jnp.int32),
        pltpu.VMEM((CHUNK,), jnp.int32),
        pltpu.VMEM((CHUNK,), jnp.int32),
        pltpu.VMEM((CHUNK, DIM), jnp.float32),
        pltpu.VMEM((CHUNK, DIM), jnp.float32),
        pltpu.VMEM_SHARED((N_PAD, DIM), jnp.float32),
        pltpu.SemaphoreType.DMA,
        pltpu.SemaphoreType.DMA,
        pltpu.SemaphoreType.DMA,
        pltpu.SemaphoreType.DMA,
    ],
)
def _scatter_kernel(
    row_hbm, col_hbm, y_hbm, out_hbm,
    row_a, col_a, row_b, col_b, buf_a, buf_b, acc_s,
    sem_ia, sem_ib, sem_ga, sem_gb,
):
    cid = lax.axis_index("c")
    sid = lax.axis_index("s")
    base = (sid * 2 * NCHUNK + cid * NCHUNK0) * CHUNK
    nch = jnp.where(cid == 0, NCHUNK0, NCHUNK1)

    def _zero(i, _):
        for k in range(DIM // 16):
            buf_a[i, pl.ds(16 * k, 16)] = jnp.zeros((16,), jnp.float32)
        return 0

    lax.fori_loop(0, CHUNK, _zero, 0)
    _zero_stripe(sid, buf_a, acc_s)
    plsc.subcore_barrier()

    def _start_idx(j, row_v, col_v, sem):
        off = base + j * CHUNK
        pltpu.async_copy(row_hbm.at[pl.ds(off, CHUNK)], row_v, sem)
        pltpu.async_copy(col_hbm.at[pl.ds(off, CHUNK)], col_v, sem)

    def _wait_idx(row_v, col_v, sem):
        pltpu.make_async_copy(row_hbm.at[pl.ds(0, CHUNK)], row_v, sem).wait()
        pltpu.make_async_copy(col_hbm.at[pl.ds(0, CHUNK)], col_v, sem).wait()

    def _wait_gather(buf, sem):
        pltpu.make_async_copy(y_hbm.at[pl.ds(0, CHUNK)], buf, sem).wait()

    _start_idx(0, row_a, col_a, sem_ia)
    _wait_idx(row_a, col_a, sem_ia)
    pltpu.async_copy(y_hbm.at[row_a], buf_a, sem_ga)
    _start_idx(1, row_b, col_b, sem_ib)

    def _pair(i, _):
        j0 = 2 * i
        j1 = j0 + 1
        _wait_idx(row_b, col_b, sem_ib)
        pltpu.async_copy(y_hbm.at[row_b], buf_b, sem_gb)
        _wait_gather(buf_a, sem_ga)
        pltpu.sync_copy(buf_a, acc_s.at[col_a], add=True)

        @pl.when(j0 + 2 < nch)
        def _():
            _start_idx(j0 + 2, row_a, col_a, sem_ia)

        _wait_gather(buf_b, sem_gb)
        pltpu.sync_copy(buf_b, acc_s.at[col_b], add=True)

        @pl.when(j0 + 2 < nch)
        def _():
            _wait_idx(row_a, col_a, sem_ia)
            pltpu.async_copy(y_hbm.at[row_a], buf_a, sem_ga)

        @pl.when(j1 + 2 < nch)
        def _():
            _start_idx(j1 + 2, row_b, col_b, sem_ib)

        return 0

    lax.fori_loop(0, nch // 2, _pair, 0)
    plsc.subcore_barrier()
    _copy_out_stripe(sid, cid, acc_s, out_hbm)


def _mm_body(x_ref, w_ref, xw_ref):
    xw_ref[...] = jnp.dot(x_ref[...], w_ref[...], preferred_element_type=jnp.float32)


def _scale_body(xw_ref, degp_ref, y_ref):
    dinv = lax.rsqrt(degp_ref[0] + degp_ref[1] + 1.0)
    y_ref[...] = dinv * xw_ref[...]


def _combine_body(p_ref, y_ref, degp_ref, out_ref):
    dinv = lax.rsqrt(degp_ref[0] + degp_ref[1] + 1.0)
    out_ref[...] = dinv * (p_ref[0] + p_ref[1] + y_ref[...])


_MM_ROWS = 1000


def kernel(node_feature, edge_index, W):
    row = edge_index[0].astype(jnp.int32)
    col = edge_index[1].astype(jnp.int32)
    pad_idx = jnp.arange(PAD, dtype=jnp.int32)
    row_p = jnp.concatenate([row, pad_idx % N_NODES])
    col_p = jnp.concatenate([col, N_NODES + (pad_idx % (N_PAD - N_NODES))])

    deg_p = _deg_kernel(col_p)

    grid = N_NODES // _MM_ROWS
    xw = pl.pallas_call(
        _mm_body,
        grid=(grid,),
        in_specs=[
            pl.BlockSpec((_MM_ROWS, DIM), lambda i: (i, 0)),
            pl.BlockSpec((DIM, DIM), lambda i: (0, 0)),
        ],
        out_specs=pl.BlockSpec((_MM_ROWS, DIM), lambda i: (i, 0)),
        out_shape=jax.ShapeDtypeStruct((N_NODES, DIM), jnp.float32),
    )(node_feature, W)

    y = pl.pallas_call(
        _scale_body,
        grid=(grid,),
        in_specs=[
            pl.BlockSpec((_MM_ROWS, DIM), lambda i: (i, 0)),
            pl.BlockSpec((NC, _MM_ROWS, DIM), lambda i: (0, i, 0)),
        ],
        out_specs=pl.BlockSpec((_MM_ROWS, DIM), lambda i: (i, 0)),
        out_shape=jax.ShapeDtypeStruct((N_NODES, DIM), jnp.float32),
    )(xw, deg_p)

    parts = _scatter_kernel(row_p, col_p, y)

    out = pl.pallas_call(
        _combine_body,
        grid=(grid,),
        in_specs=[
            pl.BlockSpec((NC, _MM_ROWS, DIM), lambda i: (0, i, 0)),
            pl.BlockSpec((_MM_ROWS, DIM), lambda i: (i, 0)),
            pl.BlockSpec((NC, _MM_ROWS, DIM), lambda i: (0, i, 0)),
        ],
        out_specs=pl.BlockSpec((_MM_ROWS, DIM), lambda i: (i, 0)),
        out_shape=jax.ShapeDtypeStruct((N_NODES, DIM), jnp.float32),
    )(parts, y, deg_p)
    return out
